# baseline (speedup 1.0000x reference)
import functools

import jax
import jax.numpy as jnp
from jax import lax
from jax.experimental import pallas as pl
from jax.experimental.pallas import tpu as pltpu
from jax.experimental.pallas import tpu_sc as plsc

NC = 2
NS = 16
NW = NC * NS
CH = 128
NB = 4


_VB = 512


def _repack_body(t_ref, o_ref):
    a = jnp.transpose(t_ref[:, : _VB // 2])
    b = jnp.transpose(t_ref[:, _VB // 2 :])
    o_ref[...] = jnp.concatenate([a, b], axis=1)


@functools.lru_cache(maxsize=None)
def _repack_build(D, V):
    brow = _VB * D // 128
    nrow = pl.cdiv(V, _VB) * brow
    return pl.pallas_call(
        _repack_body,
        grid=(pl.cdiv(V, _VB),),
        in_specs=[pl.BlockSpec((D, _VB), lambda i: (0, i))],
        out_specs=pl.BlockSpec((brow, 128), lambda i: (i, 0)),
        out_shape=jax.ShapeDtypeStruct((nrow, 128), jnp.float32),
    )


def _repack(tableT):
    D, V = tableT.shape
    return _repack_build(D, V)(tableT)


@functools.lru_cache(maxsize=None)
def _build(N, D, NSLOT):
    assert N % (NW * CH) == 0
    b_per_w = N // NW
    nch = b_per_w // CH
    ngroups = nch // NB
    assert nch % NB == 0
    mesh = plsc.VectorSubcoreMesh(core_axis_name="c", subcore_axis_name="s")

    @functools.partial(
        pl.kernel,
        out_type=jax.ShapeDtypeStruct((NSLOT, D), jnp.float32),
        mesh=mesh,
        compiler_params=pltpu.CompilerParams(use_tc_tiling_on_sc=False),
        scratch_types=[
            pltpu.VMEM((nch, CH), jnp.int32),
            pltpu.VMEM((nch, CH), jnp.int32),
            pltpu.VMEM((NB, CH, D), jnp.float32),
        ] + [pltpu.SemaphoreType.DMA] * (2 * NB),
    )
    def emb(idx_hbm, dst_hbm, table_hbm, out_hbm, idx_v, dst_v, rows_v, *sems):
        gsems, wsems = sems[:NB], sems[NB:]
        wid = lax.axis_index("s") * NC + lax.axis_index("c")
        pltpu.sync_copy(idx_hbm.at[pl.ds(wid * nch, nch)], idx_v)
        pltpu.sync_copy(dst_hbm.at[pl.ds(wid * nch, nch)], dst_v)

        for b in range(NB):
            pltpu.async_copy(table_hbm.at[idx_v.at[b]], rows_v.at[b], gsems[b])

        def group(p, carry):
            for b in range(NB):
                j = p * NB + b
                pltpu.make_async_copy(
                    table_hbm.at[idx_v.at[j]], rows_v.at[b], gsems[b]
                ).wait()
                pltpu.async_copy(
                    rows_v.at[b], out_hbm.at[dst_v.at[j]], wsems[b]
                )
                bp = (b - 1) % NB
                jp = j - 1
                jn = jp + NB

                @pl.when(jp >= 0)
                def _():
                    pltpu.make_async_copy(
                        rows_v.at[bp], out_hbm.at[dst_v.at[jp]], wsems[bp]
                    ).wait()

                @pl.when(jnp.logical_and(jp >= 0, jn < nch))
                def _():
                    pltpu.async_copy(
                        table_hbm.at[idx_v.at[jn]], rows_v.at[bp], gsems[bp]
                    )
            return carry

        lax.fori_loop(0, ngroups, group, 0)
        bl = (nch - 1) % NB
        pltpu.make_async_copy(
            rows_v.at[bl], out_hbm.at[dst_v.at[nch - 1]], wsems[bl]
        ).wait()

    return emb


def kernel(X, table):
    B, H = X.shape
    N = B * H
    D = table.shape[1]
    assert D == 64
    HP = ((H + 7) // 8) * 8
    Xi = X.astype(jnp.int32)
    HB = _VB // 2
    idx2d = (
        2 * ((Xi // _VB) * HB + Xi % HB) + (Xi % _VB) // HB
    ).reshape(N // CH, CH)
    n = jnp.arange(N, dtype=jnp.int32)
    dst2d = (2 * (HP * (n // H) + (n % H))).reshape(N // CH, CH)
    lin = _repack(table.T).reshape(-1, D)
    out = _build(N, D, B * HP * 2)(idx2d, dst2d, lin)
    return out.reshape(B, HP, 2 * D)[:, :H, :D]


# speedup vs baseline: 1.6689x; 1.6689x over previous
import functools

import jax
import jax.numpy as jnp
from jax import lax
from jax.experimental import pallas as pl
from jax.experimental.pallas import tpu as pltpu
from jax.experimental.pallas import tpu_sc as plsc

NC = 2
NS = 16
NW = NC * NS
CH = 128
NB = 8


@functools.lru_cache(maxsize=None)
def _build(N, D, NSLOT):
    assert N % (NW * CH) == 0
    b_per_w = N // NW
    nch = b_per_w // CH
    ngroups = nch // NB
    assert nch % NB == 0
    mesh = plsc.VectorSubcoreMesh(core_axis_name="c", subcore_axis_name="s")

    @functools.partial(
        pl.kernel,
        out_type=jax.ShapeDtypeStruct((NSLOT, D), jnp.float32),
        mesh=mesh,
        compiler_params=pltpu.CompilerParams(use_tc_tiling_on_sc=False),
        scratch_types=[
            pltpu.VMEM((nch, CH), jnp.int32),
            pltpu.VMEM((nch, CH), jnp.int32),
            pltpu.VMEM((NB, CH, D), jnp.float32),
        ] + [pltpu.SemaphoreType.DMA] * (2 * NB),
    )
    def emb(idx_hbm, dst_hbm, table_hbm, out_hbm, idx_v, dst_v, rows_v, *sems):
        gsems, wsems = sems[:NB], sems[NB:]
        wid = lax.axis_index("s") * NC + lax.axis_index("c")
        pltpu.sync_copy(idx_hbm.at[pl.ds(wid * nch, nch)], idx_v)
        pltpu.sync_copy(dst_hbm.at[pl.ds(wid * nch, nch)], dst_v)

        for b in range(NB):
            pltpu.async_copy(table_hbm.at[idx_v.at[b]], rows_v.at[b], gsems[b])

        def group(p, carry):
            for b in range(NB):
                j = p * NB + b
                pltpu.make_async_copy(
                    table_hbm.at[idx_v.at[j]], rows_v.at[b], gsems[b]
                ).wait()
                pltpu.async_copy(
                    rows_v.at[b], out_hbm.at[dst_v.at[j]], wsems[b]
                )
                bp = (b - 1) % NB
                jp = j - 1
                jn = jp + NB

                @pl.when(jp >= 0)
                def _():
                    pltpu.make_async_copy(
                        rows_v.at[bp], out_hbm.at[dst_v.at[jp]], wsems[bp]
                    ).wait()

                @pl.when(jnp.logical_and(jp >= 0, jn < nch))
                def _():
                    pltpu.async_copy(
                        table_hbm.at[idx_v.at[jn]], rows_v.at[bp], gsems[bp]
                    )
            return carry

        lax.fori_loop(0, ngroups, group, 0)
        bl = (nch - 1) % NB
        pltpu.make_async_copy(
            rows_v.at[bl], out_hbm.at[dst_v.at[nch - 1]], wsems[bl]
        ).wait()

    return emb


def kernel(X, table):
    B, H = X.shape
    N = B * H
    D = table.shape[1]
    assert D == 64
    HP = ((H + 7) // 8) * 8
    idx2d = X.reshape(N // CH, CH).astype(jnp.int32)
    n = jnp.arange(N, dtype=jnp.int32)
    dst2d = (2 * (HP * (n // H) + (n % H))).reshape(N // CH, CH)
    out = _build(N, D, B * HP * 2)(idx2d, dst2d, table)
    return out.reshape(B, HP, 2 * D)[:, :H, :D]
